# baseline (speedup 1.0000x reference)
import functools

import jax
import jax.numpy as jnp
from jax import lax
from jax.experimental import pallas as pl
from jax.experimental.pallas import tpu as pltpu
from jax.experimental.pallas import tpu_sc as plsc

NC = 2
NS = 16
NW = NC * NS
L = 16
C = 80


def _make_kernel(E, H):
  n_per_w = E // NW
  n_chunks = n_per_w // C
  groups = C // L
  mesh = plsc.VectorSubcoreMesh(core_axis_name="c", subcore_axis_name="s")

  @functools.partial(
      pl.kernel,
      out_type=jax.ShapeDtypeStruct((E,), jnp.float32),
      mesh=mesh,
      compiler_params=pltpu.CompilerParams(needs_layout_passes=False,
                                           use_tc_tiling_on_sc=False),
      scratch_types=[
          pltpu.VMEM((n_per_w,), jnp.int32),
          pltpu.VMEM((n_per_w,), jnp.int32),
          pltpu.VMEM((n_per_w,), jnp.int32),
          pltpu.VMEM((C, H), jnp.float32),
          pltpu.VMEM((C, H), jnp.float32),
          pltpu.VMEM((C, H), jnp.float32),
          pltpu.VMEM((C, H), jnp.float32),
          pltpu.VMEM((C, H), jnp.float32),
          pltpu.VMEM((C, H), jnp.float32),
          pltpu.VMEM((n_per_w + L,), jnp.float32),
          pltpu.SemaphoreType.DMA,
          pltpu.SemaphoreType.DMA,
          pltpu.SemaphoreType.DMA,
          pltpu.SemaphoreType.DMA,
          pltpu.SemaphoreType.DMA,
          pltpu.SemaphoreType.DMA,
      ],
  )
  def distmult(z_hbm, src_hbm, dst_hbm, et_hbm, rel_hbm, out_hbm,
               si, di, ti, sr0, dr0, rr0, sr1, dr1, rr1, ov,
               ss0, sd0, st0, ss1, sd1, st1):
    wid = lax.axis_index("s") * NC + lax.axis_index("c")
    wbase = wid * n_per_w
    mask_last = lax.iota(jnp.int32, L) == (L - 1)

    pltpu.sync_copy(src_hbm.at[pl.ds(wbase, n_per_w)], si)
    pltpu.sync_copy(dst_hbm.at[pl.ds(wbase, n_per_w)], di)
    pltpu.sync_copy(et_hbm.at[pl.ds(wbase, n_per_w)], ti)

    bufs = ((sr0, dr0, rr0, ss0, sd0, st0),
            (sr1, dr1, rr1, ss1, sd1, st1))

    def copies(c, b):
      sr, dr, rr, ss, sd, st = bufs[b]
      off = c * C
      return (
          pltpu.make_async_copy(z_hbm.at[si.at[pl.ds(off, C)]], sr, ss),
          pltpu.make_async_copy(z_hbm.at[di.at[pl.ds(off, C)]], dr, sd),
          pltpu.make_async_copy(rel_hbm.at[ti.at[pl.ds(off, C)]], rr, st),
      )

    def issue(c, b):
      @pl.when(c < n_chunks)
      def _():
        for cp in copies(c, b):
          cp.start()

    def compute(c, b):
      sr, dr, rr, _, _, _ = bufs[b]
      for cp in copies(c, b):
        cp.wait()

      def group(g, _):
        e0 = g * L
        for el in range(L):
          e = e0 + el
          t = []
          for j in range(H // L):
            o = j * L
            t.append((sr[e, pl.ds(o, L)] * dr[e, pl.ds(o, L)]) * rr[e, pl.ds(o, L)])
          while len(t) > 1:
            t = [a + b for a, b in zip(t[::2], t[1::2])]
          acc = t[0]
          plsc.store_compressed(ov.at[pl.ds(c * C + e, L)], plsc.cumsum(acc),
                                mask=mask_last)
        return _

      lax.fori_loop(0, groups, group, None)

    issue(0, 0)

    def pair(j, _):
      c0 = 2 * j
      c1 = c0 + 1
      issue(c1, 1)
      compute(c0, 0)
      issue(c0 + 2, 0)

      @pl.when(c1 < n_chunks)
      def _():
        compute(c1, 1)

      return _

    lax.fori_loop(0, (n_chunks + 1) // 2, pair, None)
    pltpu.sync_copy(ov.at[pl.ds(0, n_per_w)], out_hbm.at[pl.ds(wbase, n_per_w)])

  return distmult


def kernel(z, edge_index, edge_type, rel_emb):
  E = edge_index.shape[1]
  H = z.shape[1]
  src = edge_index[0].astype(jnp.int32)
  dst = edge_index[1].astype(jnp.int32)
  et = edge_type.astype(jnp.int32)
  return _make_kernel(E, H)(z, src, dst, et, rel_emb)


# speedup vs baseline: 1.0062x; 1.0062x over previous
import functools

import jax
import jax.numpy as jnp
from jax import lax
from jax.experimental import pallas as pl
from jax.experimental.pallas import tpu as pltpu
from jax.experimental.pallas import tpu_sc as plsc

NC = 2
NS = 16
NW = NC * NS
L = 16
C = 80


def _make_kernel(E, W):
  n_per_w = E // NW
  n_chunks = n_per_w // C
  groups = C // L
  mesh = plsc.VectorSubcoreMesh(core_axis_name="c", subcore_axis_name="s")

  @functools.partial(
      pl.kernel,
      out_type=jax.ShapeDtypeStruct((E,), jnp.float32),
      mesh=mesh,
      compiler_params=pltpu.CompilerParams(needs_layout_passes=False,
                                           use_tc_tiling_on_sc=False),
      scratch_types=[
          pltpu.VMEM((n_per_w,), jnp.int32),
          pltpu.VMEM((n_per_w,), jnp.int32),
          pltpu.VMEM((n_per_w,), jnp.int32),
          pltpu.VMEM((C, W), jnp.int32),
          pltpu.VMEM((C, W), jnp.int32),
          pltpu.VMEM((C, W), jnp.int32),
          pltpu.VMEM((C, W), jnp.int32),
          pltpu.VMEM((C, W), jnp.int32),
          pltpu.VMEM((C, W), jnp.int32),
          pltpu.VMEM((n_per_w + L,), jnp.float32),
          pltpu.SemaphoreType.DMA,
          pltpu.SemaphoreType.DMA,
          pltpu.SemaphoreType.DMA,
          pltpu.SemaphoreType.DMA,
          pltpu.SemaphoreType.DMA,
          pltpu.SemaphoreType.DMA,
      ],
  )
  def distmult(z_hbm, src_hbm, dst_hbm, et_hbm, rel_hbm, out_hbm,
               si, di, ti, sr0, dr0, rr0, sr1, dr1, rr1, ov,
               ss0, sd0, st0, ss1, sd1, st1):
    wid = lax.axis_index("s") * NC + lax.axis_index("c")
    wbase = wid * n_per_w
    mask_last = lax.iota(jnp.int32, L) == (L - 1)
    mask_hi = jnp.full((L,), jnp.int32(-65536))

    def widen(wv):
      hi = lax.bitcast_convert_type(wv & mask_hi, jnp.float32)
      lo = lax.bitcast_convert_type(lax.shift_left(wv, 16), jnp.float32)
      return lo, hi

    pltpu.sync_copy(src_hbm.at[pl.ds(wbase, n_per_w)], si)
    pltpu.sync_copy(dst_hbm.at[pl.ds(wbase, n_per_w)], di)
    pltpu.sync_copy(et_hbm.at[pl.ds(wbase, n_per_w)], ti)

    bufs = ((sr0, dr0, rr0, ss0, sd0, st0),
            (sr1, dr1, rr1, ss1, sd1, st1))

    def copies(c, b):
      sr, dr, rr, ss, sd, st = bufs[b]
      off = c * C
      return (
          pltpu.make_async_copy(z_hbm.at[si.at[pl.ds(off, C)]], sr, ss),
          pltpu.make_async_copy(z_hbm.at[di.at[pl.ds(off, C)]], dr, sd),
          pltpu.make_async_copy(rel_hbm.at[ti.at[pl.ds(off, C)]], rr, st),
      )

    def issue(c, b):
      @pl.when(c < n_chunks)
      def _():
        for cp in copies(c, b):
          cp.start()

    def compute(c, b):
      sr, dr, rr, _, _, _ = bufs[b]
      for cp in copies(c, b):
        cp.wait()

      def group(g, _):
        e0 = g * L
        for el in range(L):
          e = e0 + el
          acc = None
          for j in range(W // L):
            o = j * L
            slo, shi = widen(sr[e, pl.ds(o, L)])
            dlo, dhi = widen(dr[e, pl.ds(o, L)])
            rlo, rhi = widen(rr[e, pl.ds(o, L)])
            t = (slo * dlo) * rlo + (shi * dhi) * rhi
            acc = t if acc is None else acc + t
          plsc.store_compressed(ov.at[pl.ds(c * C + e, L)], plsc.cumsum(acc),
                                mask=mask_last)
        return _

      lax.fori_loop(0, groups, group, None)

    issue(0, 0)

    def pair(j, _):
      c0 = 2 * j
      c1 = c0 + 1
      issue(c1, 1)
      compute(c0, 0)
      issue(c0 + 2, 0)

      @pl.when(c1 < n_chunks)
      def _():
        compute(c1, 1)

      return _

    lax.fori_loop(0, (n_chunks + 1) // 2, pair, None)
    pltpu.sync_copy(ov.at[pl.ds(0, n_per_w)], out_hbm.at[pl.ds(wbase, n_per_w)])

  return distmult


def _pack_rows(t):
  tb = t.astype(jnp.bfloat16)
  return lax.bitcast_convert_type(
      tb.reshape(t.shape[0], t.shape[1] // 2, 2), jnp.int32)


def kernel(z, edge_index, edge_type, rel_emb):
  E = edge_index.shape[1]
  H = z.shape[1]
  src = edge_index[0].astype(jnp.int32)
  dst = edge_index[1].astype(jnp.int32)
  et = edge_type.astype(jnp.int32)
  return _make_kernel(E, H // 2)(_pack_rows(z), src, dst, et,
                                 _pack_rows(rel_emb))


# speedup vs baseline: 1.1068x; 1.1000x over previous
import functools

import jax
import jax.numpy as jnp
from jax import lax
from jax.experimental import pallas as pl
from jax.experimental.pallas import tpu as pltpu
from jax.experimental.pallas import tpu_sc as plsc

NC = 2
NS = 16
NW = NC * NS
L = 16
C = 80


def _make_kernel(E, W):
  n_per_w = E // NW
  n_chunks = n_per_w // C
  groups = C // L
  mesh = plsc.VectorSubcoreMesh(core_axis_name="c", subcore_axis_name="s")

  @functools.partial(
      pl.kernel,
      out_type=jax.ShapeDtypeStruct((E,), jnp.float32),
      mesh=mesh,
      compiler_params=pltpu.CompilerParams(needs_layout_passes=False,
                                           use_tc_tiling_on_sc=False),
      scratch_types=[
          pltpu.VMEM((n_per_w,), jnp.int32),
          pltpu.VMEM((n_per_w,), jnp.int32),
          pltpu.VMEM((n_per_w,), jnp.int32),
          pltpu.VMEM((C, W), jnp.int32),
          pltpu.VMEM((C, W), jnp.int32),
          pltpu.VMEM((C, W), jnp.int32),
          pltpu.VMEM((C, W), jnp.int32),
          pltpu.VMEM((C, W), jnp.int32),
          pltpu.VMEM((C, W), jnp.int32),
          pltpu.VMEM((n_per_w + L,), jnp.float32),
          pltpu.SemaphoreType.DMA,
          pltpu.SemaphoreType.DMA,
          pltpu.SemaphoreType.DMA,
          pltpu.SemaphoreType.DMA,
          pltpu.SemaphoreType.DMA,
          pltpu.SemaphoreType.DMA,
      ],
  )
  def distmult(z_hbm, src_hbm, dst_hbm, et_hbm, rel_hbm, out_hbm,
               si, di, ti, sr0, dr0, rr0, sr1, dr1, rr1, ov,
               ss0, sd0, st0, ss1, sd1, st1):
    wid = lax.axis_index("s") * NC + lax.axis_index("c")
    wbase = wid * n_per_w
    mask_last = lax.iota(jnp.int32, L) == (L - 1)
    mask_hi = jnp.full((L,), jnp.int32(-65536))

    def widen(wv):
      hi = lax.bitcast_convert_type(wv & mask_hi, jnp.float32)
      lo = lax.bitcast_convert_type(lax.shift_left(wv, 16), jnp.float32)
      return lo, hi

    pltpu.sync_copy(src_hbm.at[pl.ds(wbase, n_per_w)], si)
    pltpu.sync_copy(dst_hbm.at[pl.ds(wbase, n_per_w)], di)
    pltpu.sync_copy(et_hbm.at[pl.ds(wbase, n_per_w)], ti)

    bufs = ((sr0, dr0, rr0, ss0, sd0, st0),
            (sr1, dr1, rr1, ss1, sd1, st1))

    def copies(c, b):
      sr, dr, rr, ss, sd, st = bufs[b]
      off = c * C
      return (
          pltpu.make_async_copy(z_hbm.at[si.at[pl.ds(off, C)]], sr, ss),
          pltpu.make_async_copy(z_hbm.at[di.at[pl.ds(off, C)]], dr, sd),
          pltpu.make_async_copy(rel_hbm.at[ti.at[pl.ds(off, C)]], rr, st),
      )

    def issue(c, b):
      @pl.when(c < n_chunks)
      def _():
        for cp in copies(c, b):
          cp.start()

    def compute(c, b):
      sr, dr, rr, _, _, _ = bufs[b]
      for cp in copies(c, b):
        cp.wait()

      def group(g, _):
        e0 = g * L
        for el in range(L):
          e = e0 + el
          alo = ahi = None
          for j in range(W // L):
            o = j * L
            sb = plsc.bitcast(sr[e, pl.ds(o, L)], jnp.bfloat16)
            db = plsc.bitcast(dr[e, pl.ds(o, L)], jnp.bfloat16)
            rb = plsc.bitcast(rr[e, pl.ds(o, L)], jnp.bfloat16)
            lo, hi = widen(plsc.bitcast((sb * db) * rb, jnp.int32))
            alo = lo if alo is None else alo + lo
            ahi = hi if ahi is None else ahi + hi
          acc = alo + ahi
          plsc.store_compressed(ov.at[pl.ds(c * C + e, L)], plsc.cumsum(acc),
                                mask=mask_last)
        return _

      lax.fori_loop(0, groups, group, None)

    issue(0, 0)

    def pair(j, _):
      c0 = 2 * j
      c1 = c0 + 1
      issue(c1, 1)
      compute(c0, 0)
      issue(c0 + 2, 0)

      @pl.when(c1 < n_chunks)
      def _():
        compute(c1, 1)

      return _

    lax.fori_loop(0, (n_chunks + 1) // 2, pair, None)
    pltpu.sync_copy(ov.at[pl.ds(0, n_per_w)], out_hbm.at[pl.ds(wbase, n_per_w)])

  return distmult


def _pack_rows(t):
  tb = t.astype(jnp.bfloat16)
  return lax.bitcast_convert_type(
      tb.reshape(t.shape[0], t.shape[1] // 2, 2), jnp.int32)


def kernel(z, edge_index, edge_type, rel_emb):
  E = edge_index.shape[1]
  H = z.shape[1]
  src = edge_index[0].astype(jnp.int32)
  dst = edge_index[1].astype(jnp.int32)
  et = edge_type.astype(jnp.int32)
  return _make_kernel(E, H // 2)(_pack_rows(z), src, dst, et,
                                 _pack_rows(rel_emb))


# speedup vs baseline: 1.1068x; 1.0000x over previous
import functools

import jax
import jax.numpy as jnp
from jax import lax
from jax.experimental import pallas as pl
from jax.experimental.pallas import tpu as pltpu
from jax.experimental.pallas import tpu_sc as plsc

NC = 2
NS = 16
NW = NC * NS
L = 16
C = 80


def _make_kernel(E, W):
  n_per_w = E // NW
  n_chunks = n_per_w // C
  groups = C // L
  mesh = plsc.VectorSubcoreMesh(core_axis_name="c", subcore_axis_name="s")

  @functools.partial(
      pl.kernel,
      out_type=jax.ShapeDtypeStruct((E,), jnp.float32),
      mesh=mesh,
      compiler_params=pltpu.CompilerParams(needs_layout_passes=False,
                                           use_tc_tiling_on_sc=False),
      scratch_types=[
          pltpu.VMEM((n_per_w,), jnp.int32),
          pltpu.VMEM((n_per_w,), jnp.int32),
          pltpu.VMEM((n_per_w,), jnp.int32),
          pltpu.VMEM((C, W), jnp.int32),
          pltpu.VMEM((C, W), jnp.int32),
          pltpu.VMEM((C, W), jnp.int32),
          pltpu.VMEM((C, W), jnp.int32),
          pltpu.VMEM((C, W), jnp.int32),
          pltpu.VMEM((C, W), jnp.int32),
          pltpu.VMEM((n_per_w + L,), jnp.float32),
          pltpu.SemaphoreType.DMA,
          pltpu.SemaphoreType.DMA,
          pltpu.SemaphoreType.DMA,
          pltpu.SemaphoreType.DMA,
          pltpu.SemaphoreType.DMA,
          pltpu.SemaphoreType.DMA,
      ],
  )
  def distmult(z_hbm, src_hbm, dst_hbm, et_hbm, rel_hbm, out_hbm,
               si, di, ti, sr0, dr0, rr0, sr1, dr1, rr1, ov,
               ss0, sd0, st0, ss1, sd1, st1):
    wid = lax.axis_index("s") * NC + lax.axis_index("c")
    wbase = wid * n_per_w
    mask_last = lax.iota(jnp.int32, L) == (L - 1)
    mask_hi = jnp.full((L,), jnp.int32(-65536))

    def widen(wv):
      hi = lax.bitcast_convert_type(wv & mask_hi, jnp.float32)
      lo = lax.bitcast_convert_type(lax.shift_left(wv, 16), jnp.float32)
      return lo, hi

    pltpu.sync_copy(src_hbm.at[pl.ds(wbase, n_per_w)], si)
    pltpu.sync_copy(dst_hbm.at[pl.ds(wbase, n_per_w)], di)
    pltpu.sync_copy(et_hbm.at[pl.ds(wbase, n_per_w)], ti)

    bufs = ((sr0, dr0, rr0, ss0, sd0, st0),
            (sr1, dr1, rr1, ss1, sd1, st1))

    def copies(c, b):
      sr, dr, rr, ss, sd, st = bufs[b]
      off = c * C
      return (
          pltpu.make_async_copy(z_hbm.at[si.at[pl.ds(off, C)]], sr, ss),
          pltpu.make_async_copy(z_hbm.at[di.at[pl.ds(off, C)]], dr, sd),
          pltpu.make_async_copy(rel_hbm.at[ti.at[pl.ds(off, C)]], rr, st),
      )

    def issue(c, b):
      @pl.when(c < n_chunks)
      def _():
        for cp in copies(c, b):
          cp.start()

    def compute(c, b):
      sr, dr, rr, _, _, _ = bufs[b]
      for cp in copies(c, b):
        cp.wait()

      def group(g, _):
        e0 = g * L
        for el in range(L):
          e = e0 + el
          p = []
          for j in range(W // L):
            o = j * L
            sb = plsc.bitcast(sr[e, pl.ds(o, L)], jnp.bfloat16)
            db = plsc.bitcast(dr[e, pl.ds(o, L)], jnp.bfloat16)
            rb = plsc.bitcast(rr[e, pl.ds(o, L)], jnp.bfloat16)
            p.append((sb * db) * rb)
          while len(p) > 1:
            p = [a + b for a, b in zip(p[::2], p[1::2])]
          lo, hi = widen(plsc.bitcast(p[0], jnp.int32))
          acc = lo + hi
          plsc.store_compressed(ov.at[pl.ds(c * C + e, L)], plsc.cumsum(acc),
                                mask=mask_last)
        return _

      lax.fori_loop(0, groups, group, None)

    issue(0, 0)

    def pair(j, _):
      c0 = 2 * j
      c1 = c0 + 1
      issue(c1, 1)
      compute(c0, 0)
      issue(c0 + 2, 0)

      @pl.when(c1 < n_chunks)
      def _():
        compute(c1, 1)

      return _

    lax.fori_loop(0, (n_chunks + 1) // 2, pair, None)
    pltpu.sync_copy(ov.at[pl.ds(0, n_per_w)], out_hbm.at[pl.ds(wbase, n_per_w)])

  return distmult


def _pack_rows(t):
  tb = t.astype(jnp.bfloat16)
  return lax.bitcast_convert_type(
      tb.reshape(t.shape[0], t.shape[1] // 2, 2), jnp.int32)


def kernel(z, edge_index, edge_type, rel_emb):
  E = edge_index.shape[1]
  H = z.shape[1]
  src = edge_index[0].astype(jnp.int32)
  dst = edge_index[1].astype(jnp.int32)
  et = edge_type.astype(jnp.int32)
  return _make_kernel(E, H // 2)(_pack_rows(z), src, dst, et,
                                 _pack_rows(rel_emb))
